# baseline (speedup 1.0000x reference)
import functools

import jax
import jax.numpy as jnp
from jax import lax
from jax.experimental import pallas as pl
from jax.experimental.pallas import tpu as pltpu
from jax.experimental.pallas import tpu_sc as plsc

DP = 128


def _depad_tc(xp, D):
    N = xp.shape[0]
    BLK = 2048

    def body(x_ref, o_ref):
        o_ref[...] = x_ref[:, :D]

    return pl.pallas_call(
        body,
        grid=(N // BLK,),
        in_specs=[pl.BlockSpec((BLK, DP), lambda i: (i, 0))],
        out_specs=pl.BlockSpec((BLK, D), lambda i: (i, 0)),
        out_shape=jax.ShapeDtypeStruct((N, D), jnp.float32),
    )(xp)


def kernel(x, table):
    B, S = x.shape
    V, D = table.shape
    N = B * S

    info = plsc.get_sparse_core_info()
    NC, NS = info.num_cores, info.num_subcores
    NW = NC * NS
    CHUNK = 128
    per_w = N // NW
    n_chunks = per_w // CHUNK

    table_p = jnp.pad(table, ((0, 0), (0, DP - D)))
    idx = x.reshape(NW, n_chunks, CHUNK)
    mesh = plsc.VectorSubcoreMesh(core_axis_name="c", subcore_axis_name="s")

    @functools.partial(
        pl.kernel,
        mesh=mesh,
        out_type=jax.ShapeDtypeStruct((NW, per_w, DP), jnp.float32),
        scratch_types=[
            pltpu.VMEM((n_chunks, CHUNK), jnp.int32),
            pltpu.VMEM((CHUNK, DP), jnp.float32),
            pltpu.SemaphoreType.DMA,
        ],
    )
    def emb(idx_hbm, table_hbm, out_hbm, idx_v, rows_v, sem):
        wid = lax.axis_index("s") * NC + lax.axis_index("c")
        pltpu.sync_copy(idx_hbm.at[wid], idx_v)

        def body(c, carry):
            pltpu.async_copy(table_hbm.at[idx_v.at[c]], rows_v, sem).wait()
            pltpu.sync_copy(rows_v, out_hbm.at[wid, pl.ds(c * CHUNK, CHUNK)])
            return carry

        lax.fori_loop(0, n_chunks, body, 0)

    out = emb(idx, table_p)
    return _depad_tc(out.reshape(N, DP), D).reshape(B, S, D)


# speedup vs baseline: 1.4044x; 1.4044x over previous
import functools

import jax
import jax.numpy as jnp
from jax import lax
from jax.experimental import pallas as pl
from jax.experimental.pallas import tpu as pltpu
from jax.experimental.pallas import tpu_sc as plsc

DP = 128
NBUF = 5
K = 3


def kernel(x, table):
    B, S = x.shape
    V, D = table.shape
    N = B * S

    info = plsc.get_sparse_core_info()
    NC, NS = info.num_cores, info.num_subcores
    NW = NC * NS
    CHUNK = 128
    per_w = N // NW
    n_chunks = per_w // CHUNK
    assert n_chunks % NBUF == 0 and K < NBUF

    table_p = jnp.pad(table, ((0, 0), (0, DP - D)))
    idx = x.reshape(NW, n_chunks, CHUNK)
    mesh = plsc.VectorSubcoreMesh(core_axis_name="c", subcore_axis_name="s")

    @functools.partial(
        pl.kernel,
        mesh=mesh,
        out_type=jax.ShapeDtypeStruct((NW, per_w, DP), jnp.float32),
        scratch_types=[
            pltpu.VMEM((n_chunks, CHUNK), jnp.int32),
            pltpu.VMEM((NBUF, CHUNK, DP), jnp.float32),
        ]
        + [pltpu.SemaphoreType.DMA] * (2 * NBUF),
        compiler_params=pltpu.CompilerParams(use_tc_tiling_on_sc=True),
    )
    def emb(idx_hbm, table_hbm, out_hbm, idx_v, rows_v, *sems):
        gsem, wsem = sems[:NBUF], sems[NBUF:]
        wid = lax.axis_index("s") * NC + lax.axis_index("c")
        pltpu.sync_copy(idx_hbm.at[wid], idx_v)

        def gather(c, b, sem):
            return pltpu.make_async_copy(
                table_hbm.at[idx_v.at[c]], rows_v.at[b], sem)

        def write(c, b, sem):
            return pltpu.make_async_copy(
                rows_v.at[b], out_hbm.at[wid, pl.ds(c * CHUNK, CHUNK)], sem)

        for j in range(K):
            gather(j, j, gsem[j]).start()

        def group(g, carry):
            for b in range(NBUF):
                c = g * NBUF + b
                bk = (b + K) % NBUF

                @pl.when(c < n_chunks - K)
                def _issue():
                    @pl.when(c >= NBUF - K)
                    def _release():
                        write(c + K - NBUF, bk, wsem[bk]).wait()

                    gather(c + K, bk, gsem[bk]).start()

                gather(c, b, gsem[b]).wait()
                write(c, b, wsem[b]).start()
            return carry

        lax.fori_loop(0, n_chunks // NBUF, group, 0)
        for j in range(NBUF):
            c = n_chunks - NBUF + j
            write(c, c % NBUF, wsem[c % NBUF]).wait()

    out = emb(idx, table_p)
    return out.reshape(N, DP)[:, :D].reshape(B, S, D)
